# baseline (speedup 1.0000x reference)
import functools

import jax
import jax.numpy as jnp
from jax import lax
from jax.experimental import pallas as pl
from jax.experimental.pallas import tpu as pltpu
from jax.experimental.pallas import tpu_sc as plsc

_B, _S, _D = 1024, 50, 128
_M, _NQ = 100000, 16
_K = 3
_DM = 2 * _NQ

_MT = 1024
_NMT = (_M + _MT - 1) // _MT
_NEG = -1.0


def _encoder_body(q_ref, w1_ref, b1_ref, w2_ref, b2_ref, qq_ref):
    acc = q_ref[:, 0, :]
    for i in range(1, _S):
        acc = acc + q_ref[:, i, :]
    q = acc * (1.0 / _S)
    h = jnp.tanh(
        jnp.dot(q, w1_ref[...], preferred_element_type=jnp.float32)
        + b1_ref[...]
    )
    qq_ref[...] = (
        jnp.dot(h, w2_ref[...], preferred_element_type=jnp.float32)
        + b2_ref[...]
    )


def _encoder(query, w1, b1, w2, b2):
    bt = 256
    return pl.pallas_call(
        _encoder_body,
        grid=(_B // bt,),
        in_specs=[
            pl.BlockSpec((bt, _S, _D), lambda i: (i, 0, 0)),
            pl.BlockSpec((_D, _D), lambda i: (0, 0)),
            pl.BlockSpec((1, _D), lambda i: (0, 0)),
            pl.BlockSpec((_D, _DM), lambda i: (0, 0)),
            pl.BlockSpec((1, _DM), lambda i: (0, 0)),
        ],
        out_specs=pl.BlockSpec((bt, _DM), lambda i: (i, 0)),
        out_shape=jax.ShapeDtypeStruct((_B, _DM), jnp.float32),
    )(query, w1, b1, w2, b2)


_LANES = 128
_RB = 32
_NRB = _B // _RB


def _insert3(t0, t1, t2, i0, i1, i2, v, vi):
    b = v > t0
    nt0 = jnp.where(b, v, t0)
    c = jnp.where(b, t0, v)
    ni0 = jnp.where(b, vi, i0)
    ci = jnp.where(b, i0, vi)
    b = c > t1
    nt1 = jnp.where(b, c, t1)
    c2 = jnp.where(b, t1, c)
    ni1 = jnp.where(b, ci, i1)
    ci2 = jnp.where(b, i1, ci)
    b = c2 > t2
    nt2 = jnp.where(b, c2, t2)
    ni2 = jnp.where(b, ci2, i2)
    return nt0, nt1, nt2, ni0, ni1, ni2


def _topk_body(qq_ref, mem_ref, out_ref, s_ref, t0r, t1r, t2r, i0r, i1r, i2r):
    j = pl.program_id(0)

    @pl.when(j == 0)
    def _init():
        for tr in (t0r, t1r, t2r):
            tr[...] = jnp.full(tr.shape, _NEG, jnp.float32)
        for ir in (i0r, i1r, i2r):
            ir[...] = jnp.zeros(ir.shape, jnp.float32)

    ov = lax.dot_general(
        qq_ref[...], mem_ref[...],
        (((1,), (1,)), ((), ())),
        preferred_element_type=jnp.float32,
    )
    s_ref[...] = ov * ov
    base = j * _MT

    @pl.when(j == _NMT - 1)
    def _mask_tail():
        col = lax.broadcasted_iota(jnp.int32, (_B, _MT), 1)
        s_ref[...] = jnp.where(col < _M - base, s_ref[...], _NEG)

    lane = lax.broadcasted_iota(jnp.int32, (_RB, _LANES), 1).astype(jnp.float32)
    fbase = base.astype(jnp.float32)

    def rb_step(rb, _):
        rows = pl.ds(rb * _RB, _RB)
        st = (t0r[rows, :], t1r[rows, :], t2r[rows, :],
              i0r[rows, :], i1r[rows, :], i2r[rows, :])
        for k in range(_MT // _LANES):
            v = s_ref[rows, k * _LANES : (k + 1) * _LANES]
            vi = lane + (fbase + float(k * _LANES))
            st = _insert3(*st, v, vi)
        t0r[rows, :], t1r[rows, :], t2r[rows, :] = st[0], st[1], st[2]
        i0r[rows, :], i1r[rows, :], i2r[rows, :] = st[3], st[4], st[5]
        return 0

    lax.fori_loop(0, _NRB, rb_step, 0)

    @pl.when(j == _NMT - 1)
    def _merge():
        t0, t1, t2 = t0r[...], t1r[...], t2r[...]
        i0, i1, i2 = i0r[...], i1r[...], i2r[...]
        lane_b = lax.broadcasted_iota(jnp.int32, (_B, _LANES), 1)
        for r in range(_K):
            m = jnp.max(t0, axis=1, keepdims=True)
            lsel = jnp.min(jnp.where(t0 == m, lane_b, _LANES), axis=1,
                           keepdims=True)
            hit = lane_b == lsel
            out_ref[:, 2 - r : 3 - r] = jnp.sum(
                jnp.where(hit, i0, 0.0), axis=1, keepdims=True
            ).astype(jnp.int32)
            t0 = jnp.where(hit, t1, t0)
            i0 = jnp.where(hit, i1, i0)
            t1 = jnp.where(hit, t2, t1)
            i1 = jnp.where(hit, i2, i1)
            t2 = jnp.where(hit, _NEG - 1.0, t2)


def _topk(qq, mem):
    return pl.pallas_call(
        _topk_body,
        grid=(_NMT,),
        in_specs=[
            pl.BlockSpec((_B, _DM), lambda j: (0, 0)),
            pl.BlockSpec((_MT, _DM), lambda j: (j, 0)),
        ],
        out_specs=pl.BlockSpec((_B, _K), lambda j: (0, 0)),
        out_shape=jax.ShapeDtypeStruct((_B, _K), jnp.int32),
        scratch_shapes=[pltpu.VMEM((_B, _MT), jnp.float32)]
        + [pltpu.VMEM((_B, _LANES), jnp.float32)] * 6,
    )(qq, mem)


_NC, _NS = 2, 16
_NW = _NC * _NS
_NB = _B * _K
_BPW = _NB // _NW


def _sc_gather_body(table_hbm, idx_hbm, out_hbm, idx_v, rows_v, sem):
    wid = lax.axis_index("s") * _NC + lax.axis_index("c")
    base = wid * _BPW
    pltpu.sync_copy(idx_hbm.at[pl.ds(base, _BPW)], idx_v)
    pltpu.async_copy(table_hbm.at[idx_v], rows_v, sem).wait()
    pltpu.sync_copy(rows_v, out_hbm.at[pl.ds(base, _BPW)])


def _sc_gather(table, idx):
    run = pl.kernel(
        _sc_gather_body,
        mesh=plsc.VectorSubcoreMesh(
            core_axis_name="c", subcore_axis_name="s", num_cores=_NC
        ),
        out_type=jax.ShapeDtypeStruct((_NB, _DM), jnp.float32),
        scratch_types=[
            pltpu.VMEM((_BPW,), jnp.int32),
            pltpu.VMEM((_BPW, _DM), jnp.float32),
            pltpu.SemaphoreType.DMA,
        ],
        compiler_params=pltpu.CompilerParams(use_tc_tiling_on_sc=False),
    )
    return run(table, idx)


def _mlp_body(g_ref, w3_ref, b3_ref, w4_ref, b4_ref, out_ref):
    x = g_ref[...][:, :_NQ]
    h = jnp.tanh(
        jnp.dot(x, w3_ref[...], preferred_element_type=jnp.float32)
        + b3_ref[...]
    )
    out_ref[...] = (
        jnp.dot(h, w4_ref[...], preferred_element_type=jnp.float32)
        + b4_ref[...]
    )


def _mlp(g, w3, b3, w4, b4):
    return pl.pallas_call(
        _mlp_body,
        in_specs=[
            pl.BlockSpec((_NB, _DM), lambda: (0, 0)),
            pl.BlockSpec((_NQ, _NQ), lambda: (0, 0)),
            pl.BlockSpec((1, _NQ), lambda: (0, 0)),
            pl.BlockSpec((_NQ, _D), lambda: (0, 0)),
            pl.BlockSpec((1, _D), lambda: (0, 0)),
        ],
        out_specs=pl.BlockSpec((_NB, _D), lambda: (0, 0)),
        out_shape=jax.ShapeDtypeStruct((_NB, _D), jnp.float32),
    )(g, w3, b3, w4, b4)


def kernel(query, memory_states, W1, b1, W2, b2, W3, b3, W4, b4):
    qq = _encoder(query, W1, b1.reshape(1, _D), W2, b2.reshape(1, _DM))
    idx = _topk(qq, memory_states)
    gathered = _sc_gather(memory_states, idx.reshape(_NB))
    feats = _mlp(
        gathered, W3, b3.reshape(1, _NQ), W4, b4.reshape(1, _D)
    )
    return feats.reshape(_B, _K, _D)


# speedup vs baseline: 1.0387x; 1.0387x over previous
import functools

import jax
import jax.numpy as jnp
from jax import lax
from jax.experimental import pallas as pl
from jax.experimental.pallas import tpu as pltpu
from jax.experimental.pallas import tpu_sc as plsc

_B, _S, _D = 1024, 50, 128
_M, _NQ = 100000, 16
_K = 3
_DM = 2 * _NQ

_MT = 1024
_NMT = (_M + _MT - 1) // _MT
_NEG = -1.0


def _encoder_body(q_ref, w1_ref, b1_ref, w2_ref, b2_ref, qq_ref):
    acc = q_ref[:, 0, :]
    for i in range(1, _S):
        acc = acc + q_ref[:, i, :]
    q = acc * (1.0 / _S)
    h = jnp.tanh(
        jnp.dot(q, w1_ref[...], preferred_element_type=jnp.float32)
        + b1_ref[...]
    )
    qq_ref[...] = (
        jnp.dot(h, w2_ref[...], preferred_element_type=jnp.float32)
        + b2_ref[...]
    )


def _encoder(query, w1, b1, w2, b2):
    bt = 256
    return pl.pallas_call(
        _encoder_body,
        grid=(_B // bt,),
        in_specs=[
            pl.BlockSpec((bt, _S, _D), lambda i: (i, 0, 0)),
            pl.BlockSpec((_D, _D), lambda i: (0, 0)),
            pl.BlockSpec((1, _D), lambda i: (0, 0)),
            pl.BlockSpec((_D, _DM), lambda i: (0, 0)),
            pl.BlockSpec((1, _DM), lambda i: (0, 0)),
        ],
        out_specs=pl.BlockSpec((bt, _DM), lambda i: (i, 0)),
        out_shape=jax.ShapeDtypeStruct((_B, _DM), jnp.float32),
    )(query, w1, b1, w2, b2)


_LANES = 128
_RB = 16
_NRB = _B // _RB


def _insert3(t0, t1, t2, i0, i1, i2, v, vi):
    b = v > t0
    nt0 = jnp.where(b, v, t0)
    c = jnp.where(b, t0, v)
    ni0 = jnp.where(b, vi, i0)
    ci = jnp.where(b, i0, vi)
    b = c > t1
    nt1 = jnp.where(b, c, t1)
    c2 = jnp.where(b, t1, c)
    ni1 = jnp.where(b, ci, i1)
    ci2 = jnp.where(b, i1, ci)
    b = c2 > t2
    nt2 = jnp.where(b, c2, t2)
    ni2 = jnp.where(b, ci2, i2)
    return nt0, nt1, nt2, ni0, ni1, ni2


def _topk_body(qq_ref, mem_ref, out_ref, s_ref, t0r, t1r, t2r, i0r, i1r, i2r):
    j = pl.program_id(0)

    @pl.when(j == 0)
    def _init():
        for tr in (t0r, t1r, t2r):
            tr[...] = jnp.full(tr.shape, _NEG, jnp.float32)
        for ir in (i0r, i1r, i2r):
            ir[...] = jnp.zeros(ir.shape, jnp.float32)

    ov = lax.dot_general(
        qq_ref[...], mem_ref[...],
        (((1,), (1,)), ((), ())),
        preferred_element_type=jnp.float32,
    )
    s_ref[...] = ov * ov
    base = j * _MT

    @pl.when(j == _NMT - 1)
    def _mask_tail():
        col = lax.broadcasted_iota(jnp.int32, (_B, _MT), 1)
        s_ref[...] = jnp.where(col < _M - base, s_ref[...], _NEG)

    lane = lax.broadcasted_iota(jnp.int32, (_RB, _LANES), 1).astype(jnp.float32)
    fbase = base.astype(jnp.float32)

    def rb_step(rb, _):
        rows = pl.ds(rb * _RB, _RB)
        st = (t0r[rows, :], t1r[rows, :], t2r[rows, :],
              i0r[rows, :], i1r[rows, :], i2r[rows, :])
        for k in range(_MT // _LANES):
            v = s_ref[rows, k * _LANES : (k + 1) * _LANES]
            vi = lane + (fbase + float(k * _LANES))
            st = _insert3(*st, v, vi)
        t0r[rows, :], t1r[rows, :], t2r[rows, :] = st[0], st[1], st[2]
        i0r[rows, :], i1r[rows, :], i2r[rows, :] = st[3], st[4], st[5]
        return 0

    lax.fori_loop(0, _NRB, rb_step, 0)

    @pl.when(j == _NMT - 1)
    def _merge():
        t0, t1, t2 = t0r[...], t1r[...], t2r[...]
        i0, i1, i2 = i0r[...], i1r[...], i2r[...]
        lane_b = lax.broadcasted_iota(jnp.int32, (_B, _LANES), 1)
        for r in range(_K):
            m = jnp.max(t0, axis=1, keepdims=True)
            lsel = jnp.min(jnp.where(t0 == m, lane_b, _LANES), axis=1,
                           keepdims=True)
            hit = lane_b == lsel
            out_ref[:, 2 - r : 3 - r] = jnp.sum(
                jnp.where(hit, i0, 0.0), axis=1, keepdims=True
            ).astype(jnp.int32)
            t0 = jnp.where(hit, t1, t0)
            i0 = jnp.where(hit, i1, i0)
            t1 = jnp.where(hit, t2, t1)
            i1 = jnp.where(hit, i2, i1)
            t2 = jnp.where(hit, _NEG - 1.0, t2)


def _topk(qq, mem):
    return pl.pallas_call(
        _topk_body,
        grid=(_NMT,),
        in_specs=[
            pl.BlockSpec((_B, _DM), lambda j: (0, 0)),
            pl.BlockSpec((_MT, _DM), lambda j: (j, 0)),
        ],
        out_specs=pl.BlockSpec((_B, _K), lambda j: (0, 0)),
        out_shape=jax.ShapeDtypeStruct((_B, _K), jnp.int32),
        scratch_shapes=[pltpu.VMEM((_B, _MT), jnp.float32)]
        + [pltpu.VMEM((_B, _LANES), jnp.float32)] * 6,
    )(qq, mem)


_NC, _NS = 2, 16
_NW = _NC * _NS
_NB = _B * _K
_BPW = _NB // _NW


def _sc_gather_body(table_hbm, idx_hbm, out_hbm, idx_v, rows_v, sem):
    wid = lax.axis_index("s") * _NC + lax.axis_index("c")
    base = wid * _BPW
    pltpu.sync_copy(idx_hbm.at[pl.ds(base, _BPW)], idx_v)
    pltpu.async_copy(table_hbm.at[idx_v], rows_v, sem).wait()
    pltpu.sync_copy(rows_v, out_hbm.at[pl.ds(base, _BPW)])


def _sc_gather(table, idx):
    run = pl.kernel(
        _sc_gather_body,
        mesh=plsc.VectorSubcoreMesh(
            core_axis_name="c", subcore_axis_name="s", num_cores=_NC
        ),
        out_type=jax.ShapeDtypeStruct((_NB, _DM), jnp.float32),
        scratch_types=[
            pltpu.VMEM((_BPW,), jnp.int32),
            pltpu.VMEM((_BPW, _DM), jnp.float32),
            pltpu.SemaphoreType.DMA,
        ],
        compiler_params=pltpu.CompilerParams(use_tc_tiling_on_sc=False),
    )
    return run(table, idx)


def _mlp_body(g_ref, w3_ref, b3_ref, w4_ref, b4_ref, out_ref):
    x = g_ref[...][:, :_NQ]
    h = jnp.tanh(
        jnp.dot(x, w3_ref[...], preferred_element_type=jnp.float32)
        + b3_ref[...]
    )
    out_ref[...] = (
        jnp.dot(h, w4_ref[...], preferred_element_type=jnp.float32)
        + b4_ref[...]
    )


def _mlp(g, w3, b3, w4, b4):
    return pl.pallas_call(
        _mlp_body,
        in_specs=[
            pl.BlockSpec((_NB, _DM), lambda: (0, 0)),
            pl.BlockSpec((_NQ, _NQ), lambda: (0, 0)),
            pl.BlockSpec((1, _NQ), lambda: (0, 0)),
            pl.BlockSpec((_NQ, _D), lambda: (0, 0)),
            pl.BlockSpec((1, _D), lambda: (0, 0)),
        ],
        out_specs=pl.BlockSpec((_NB, _D), lambda: (0, 0)),
        out_shape=jax.ShapeDtypeStruct((_NB, _D), jnp.float32),
    )(g, w3, b3, w4, b4)


def kernel(query, memory_states, W1, b1, W2, b2, W3, b3, W4, b4):
    qq = _encoder(query, W1, b1.reshape(1, _D), W2, b2.reshape(1, _DM))
    idx = _topk(qq, memory_states)
    gathered = _sc_gather(memory_states, idx.reshape(_NB))
    feats = _mlp(
        gathered, W3, b3.reshape(1, _NQ), W4, b4.reshape(1, _D)
    )
    return feats.reshape(_B, _K, _D)


# speedup vs baseline: 1.1087x; 1.0674x over previous
import functools

import jax
import jax.numpy as jnp
from jax import lax
from jax.experimental import pallas as pl
from jax.experimental.pallas import tpu as pltpu
from jax.experimental.pallas import tpu_sc as plsc

_B, _S, _D = 1024, 50, 128
_M, _NQ = 100000, 16
_K = 3
_DM = 2 * _NQ

_MT = 2048
_NMT = (_M + _MT - 1) // _MT
_NEG = -1.0


def _encoder_body(q_ref, w1_ref, b1_ref, w2_ref, b2_ref, qq_ref):
    acc = q_ref[:, 0, :]
    for i in range(1, _S):
        acc = acc + q_ref[:, i, :]
    q = acc * (1.0 / _S)
    h = jnp.tanh(
        jnp.dot(q, w1_ref[...], preferred_element_type=jnp.float32)
        + b1_ref[...]
    )
    qq_ref[...] = (
        jnp.dot(h, w2_ref[...], preferred_element_type=jnp.float32)
        + b2_ref[...]
    )


def _encoder(query, w1, b1, w2, b2):
    bt = 256
    return pl.pallas_call(
        _encoder_body,
        grid=(_B // bt,),
        in_specs=[
            pl.BlockSpec((bt, _S, _D), lambda i: (i, 0, 0)),
            pl.BlockSpec((_D, _D), lambda i: (0, 0)),
            pl.BlockSpec((1, _D), lambda i: (0, 0)),
            pl.BlockSpec((_D, _DM), lambda i: (0, 0)),
            pl.BlockSpec((1, _DM), lambda i: (0, 0)),
        ],
        out_specs=pl.BlockSpec((bt, _DM), lambda i: (i, 0)),
        out_shape=jax.ShapeDtypeStruct((_B, _DM), jnp.float32),
    )(query, w1, b1, w2, b2)


_LANES = 128
_RB = 8
_UNROLL = 2
_NRB = _B // (_RB * _UNROLL)


def _insert3(t0, t1, t2, i0, i1, i2, v, vi):
    b = v > t0
    nt0 = jnp.where(b, v, t0)
    c = jnp.where(b, t0, v)
    ni0 = jnp.where(b, vi, i0)
    ci = jnp.where(b, i0, vi)
    b = c > t1
    nt1 = jnp.where(b, c, t1)
    c2 = jnp.where(b, t1, c)
    ni1 = jnp.where(b, ci, i1)
    ci2 = jnp.where(b, i1, ci)
    b = c2 > t2
    nt2 = jnp.where(b, c2, t2)
    ni2 = jnp.where(b, ci2, i2)
    return nt0, nt1, nt2, ni0, ni1, ni2


def _topk_body(qq_ref, mem_ref, out_ref, s_ref, t0r, t1r, t2r, i0r, i1r, i2r):
    j = pl.program_id(0)

    @pl.when(j == 0)
    def _init():
        for tr in (t0r, t1r, t2r):
            tr[...] = jnp.full(tr.shape, _NEG, jnp.float32)
        for ir in (i0r, i1r, i2r):
            ir[...] = jnp.zeros(ir.shape, jnp.float32)

    ov = lax.dot_general(
        qq_ref[...], mem_ref[...],
        (((1,), (1,)), ((), ())),
        preferred_element_type=jnp.float32,
    )
    s_ref[...] = ov * ov
    base = j * _MT

    @pl.when(j == _NMT - 1)
    def _mask_tail():
        col = lax.broadcasted_iota(jnp.int32, (_B, _MT), 1)
        s_ref[...] = jnp.where(col < _M - base, s_ref[...], _NEG)

    lane = lax.broadcasted_iota(jnp.int32, (_RB, _LANES), 1).astype(jnp.float32)
    fbase = base.astype(jnp.float32)

    def rb_step(rbo, _):
        for u in range(_UNROLL):
            rows = pl.ds((rbo * _UNROLL + u) * _RB, _RB)
            st = (t0r[rows, :], t1r[rows, :], t2r[rows, :],
                  i0r[rows, :], i1r[rows, :], i2r[rows, :])
            for k in range(_MT // _LANES):
                v = s_ref[rows, k * _LANES : (k + 1) * _LANES]
                vi = lane + (fbase + float(k * _LANES))
                st = _insert3(*st, v, vi)
            t0r[rows, :], t1r[rows, :], t2r[rows, :] = st[0], st[1], st[2]
            i0r[rows, :], i1r[rows, :], i2r[rows, :] = st[3], st[4], st[5]
        return 0

    lax.fori_loop(0, _NRB, rb_step, 0)

    @pl.when(j == _NMT - 1)
    def _merge():
        t0, t1, t2 = t0r[...], t1r[...], t2r[...]
        i0, i1, i2 = i0r[...], i1r[...], i2r[...]
        lane_b = lax.broadcasted_iota(jnp.int32, (_B, _LANES), 1)
        for r in range(_K):
            m = jnp.max(t0, axis=1, keepdims=True)
            lsel = jnp.min(jnp.where(t0 == m, lane_b, _LANES), axis=1,
                           keepdims=True)
            hit = lane_b == lsel
            out_ref[:, 2 - r : 3 - r] = jnp.sum(
                jnp.where(hit, i0, 0.0), axis=1, keepdims=True
            ).astype(jnp.int32)
            t0 = jnp.where(hit, t1, t0)
            i0 = jnp.where(hit, i1, i0)
            t1 = jnp.where(hit, t2, t1)
            i1 = jnp.where(hit, i2, i1)
            t2 = jnp.where(hit, _NEG - 1.0, t2)


def _topk(qq, mem):
    return pl.pallas_call(
        _topk_body,
        grid=(_NMT,),
        in_specs=[
            pl.BlockSpec((_B, _DM), lambda j: (0, 0)),
            pl.BlockSpec((_MT, _DM), lambda j: (j, 0)),
        ],
        out_specs=pl.BlockSpec((_B, _K), lambda j: (0, 0)),
        out_shape=jax.ShapeDtypeStruct((_B, _K), jnp.int32),
        scratch_shapes=[pltpu.VMEM((_B, _MT), jnp.float32)]
        + [pltpu.VMEM((_B, _LANES), jnp.float32)] * 6,
    )(qq, mem)


_NC, _NS = 2, 16
_NW = _NC * _NS
_NB = _B * _K
_BPW = _NB // _NW


def _sc_gather_body(table_hbm, idx_hbm, out_hbm, idx_v, rows_v, sem):
    wid = lax.axis_index("s") * _NC + lax.axis_index("c")
    base = wid * _BPW
    pltpu.sync_copy(idx_hbm.at[pl.ds(base, _BPW)], idx_v)
    pltpu.async_copy(table_hbm.at[idx_v], rows_v, sem).wait()
    pltpu.sync_copy(rows_v, out_hbm.at[pl.ds(base, _BPW)])


def _sc_gather(table, idx):
    run = pl.kernel(
        _sc_gather_body,
        mesh=plsc.VectorSubcoreMesh(
            core_axis_name="c", subcore_axis_name="s", num_cores=_NC
        ),
        out_type=jax.ShapeDtypeStruct((_NB, _DM), jnp.float32),
        scratch_types=[
            pltpu.VMEM((_BPW,), jnp.int32),
            pltpu.VMEM((_BPW, _DM), jnp.float32),
            pltpu.SemaphoreType.DMA,
        ],
        compiler_params=pltpu.CompilerParams(use_tc_tiling_on_sc=False),
    )
    return run(table, idx)


def _mlp_body(g_ref, w3_ref, b3_ref, w4_ref, b4_ref, out_ref):
    x = g_ref[...][:, :_NQ]
    h = jnp.tanh(
        jnp.dot(x, w3_ref[...], preferred_element_type=jnp.float32)
        + b3_ref[...]
    )
    out_ref[...] = (
        jnp.dot(h, w4_ref[...], preferred_element_type=jnp.float32)
        + b4_ref[...]
    )


def _mlp(g, w3, b3, w4, b4):
    return pl.pallas_call(
        _mlp_body,
        in_specs=[
            pl.BlockSpec((_NB, _DM), lambda: (0, 0)),
            pl.BlockSpec((_NQ, _NQ), lambda: (0, 0)),
            pl.BlockSpec((1, _NQ), lambda: (0, 0)),
            pl.BlockSpec((_NQ, _D), lambda: (0, 0)),
            pl.BlockSpec((1, _D), lambda: (0, 0)),
        ],
        out_specs=pl.BlockSpec((_NB, _D), lambda: (0, 0)),
        out_shape=jax.ShapeDtypeStruct((_NB, _D), jnp.float32),
    )(g, w3, b3, w4, b4)


def kernel(query, memory_states, W1, b1, W2, b2, W3, b3, W4, b4):
    qq = _encoder(query, W1, b1.reshape(1, _D), W2, b2.reshape(1, _DM))
    idx = _topk(qq, memory_states)
    gathered = _sc_gather(memory_states, idx.reshape(_NB))
    feats = _mlp(
        gathered, W3, b3.reshape(1, _NQ), W4, b4.reshape(1, _D)
    )
    return feats.reshape(_B, _K, _D)
